# baseline (speedup 1.0000x reference)
import jax
import jax.numpy as jnp
from jax import lax
from jax.experimental import pallas as pl
from jax.experimental.pallas import tpu as pltpu
from jax.experimental.pallas import tpu_sc as plsc

N_MID = 1000000
N_CAT = 100000
EMBED_DIM = 32
BATCH = 16384
MAX_LEN = 200

NW = 32
BW = BATCH // NW
D2 = 2 * EMBED_DIM
DBLK = D2 // 8
BBLK = BW // 128


def _body(mid_idxT, cat_idxT, mid_table, cat_table, out_hbm,
          midx_v, cidx_v, mrows_v, crows_v, tbuf_v, gsem, wsem):
    wid = lax.axis_index("c") * 16 + lax.axis_index("s")
    b0 = wid * BW
    bb0 = wid * BBLK
    lane = lax.iota(jnp.int32, 16)

    def chunk(l, _):
        pltpu.sync_copy(mid_idxT.at[l, pl.ds(b0, BW)], midx_v)
        pltpu.sync_copy(cat_idxT.at[l, pl.ds(b0, BW)], cidx_v)
        gathers = []
        for s in range(BW // 128):
            cm = pltpu.make_async_copy(
                mid_table.at[midx_v.at[pl.ds(s * 128, 128)]],
                mrows_v.at[pl.ds(s * 128, 128), :], gsem)
            cc = pltpu.make_async_copy(
                cat_table.at[cidx_v.at[pl.ds(s * 128, 128)]],
                crows_v.at[pl.ds(s * 128, 128), :], gsem)
            cm.start()
            cc.start()
            gathers.append(cm)
            gathers.append(cc)
        for c in gathers:
            c.wait()

        @plsc.parallel_loop(0, BW // 16, unroll=4)
        def transpose_group(g):
            rows = g * 16 + lane
            for d in range(EMBED_DIM):
                col = jnp.full((16,), d, jnp.int32)
                tbuf_v[d, pl.ds(g * 16, 16)] = plsc.load_gather(
                    mrows_v, [rows, col])
                tbuf_v[EMBED_DIM + d, pl.ds(g * 16, 16)] = plsc.load_gather(
                    crows_v, [rows, col])

        writes = []
        for db in range(DBLK):
            for bb in range(BBLK):
                w = pltpu.make_async_copy(
                    tbuf_v.at[pl.ds(db * 8, 8), pl.ds(bb * 128, 128)],
                    out_hbm.at[l, db, bb0 + bb, :, :], wsem)
                w.start()
                writes.append(w)
        for c in writes:
            c.wait()
        return ()

    lax.fori_loop(0, MAX_LEN, chunk, (), unroll=False)


@jax.jit
def _run(mid_idxT, cat_idxT, mid_table, cat_table):
    mesh = plsc.VectorSubcoreMesh(core_axis_name="c", subcore_axis_name="s")
    f = pl.kernel(
        _body,
        out_type=jax.ShapeDtypeStruct(
            (MAX_LEN, DBLK, BATCH // 128, 8, 128), jnp.float32),
        mesh=mesh,
        scratch_types=[
            pltpu.VMEM((BW,), jnp.int32),
            pltpu.VMEM((BW,), jnp.int32),
            pltpu.VMEM((BW, EMBED_DIM), jnp.float32),
            pltpu.VMEM((BW, EMBED_DIM), jnp.float32),
            pltpu.VMEM((D2, BW), jnp.float32),
            pltpu.SemaphoreType.DMA,
            pltpu.SemaphoreType.DMA,
        ],
        compiler_params=pltpu.CompilerParams(use_tc_tiling_on_sc=False,
                                             needs_layout_passes=False,
                                             disable_bounds_checks=True),
    )
    return f(mid_idxT, cat_idxT, mid_table, cat_table)


def kernel(mid_his_input, cat_his_input, mid_table, cat_table):
    mid_idxT = mid_his_input.astype(jnp.int32).T
    cat_idxT = cat_his_input.astype(jnp.int32).T
    out5 = _run(mid_idxT, cat_idxT, mid_table, cat_table)
    t = jnp.transpose(out5, (2, 4, 0, 1, 3))
    return t.reshape(BATCH, MAX_LEN, D2)


# speedup vs baseline: 3.0430x; 3.0430x over previous
import jax
import jax.numpy as jnp
from jax import lax
from jax.experimental import pallas as pl
from jax.experimental.pallas import tpu as pltpu
from jax.experimental.pallas import tpu_sc as plsc

N_MID = 1000000
N_CAT = 100000
EMBED_DIM = 32
BATCH = 16384
MAX_LEN = 200

NW = 32
BW = BATCH // NW
D2 = 2 * EMBED_DIM
DBLK = D2 // 8
BBLK = BW // 128


def _body(mid_idxT, cat_idxT, mid_table, cat_table, out_hbm,
          midx_v, cidx_v, mrows_v, crows_v, tbuf_v, gsem, wsem):
    wid = lax.axis_index("c") * 16 + lax.axis_index("s")
    b0 = wid * BW
    bb0 = wid * BBLK
    lane = lax.iota(jnp.int32, 16)

    def chunk(l, _):
        pltpu.sync_copy(mid_idxT.at[l, pl.ds(b0, BW)], midx_v)
        pltpu.sync_copy(cat_idxT.at[l, pl.ds(b0, BW)], cidx_v)
        gathers = []
        for s in range(BW // 128):
            cm = pltpu.make_async_copy(
                mid_table.at[midx_v.at[pl.ds(s * 128, 128)]],
                mrows_v.at[pl.ds(s * 128, 128), :], gsem)
            cc = pltpu.make_async_copy(
                cat_table.at[cidx_v.at[pl.ds(s * 128, 128)]],
                crows_v.at[pl.ds(s * 128, 128), :], gsem)
            cm.start()
            cc.start()
            gathers.append(cm)
            gathers.append(cc)
        for c in gathers:
            c.wait()

        r00 = lane
        r16 = lane + 16
        r32 = lane + 32
        r48 = lane + 48

        @plsc.parallel_loop(0, BW, unroll=4)
        def transpose_row(b):
            colb = jnp.zeros((16,), jnp.int32) + b
            plsc.store_scatter(tbuf_v, [r00, colb], mrows_v[b, pl.ds(0, 16)])
            plsc.store_scatter(tbuf_v, [r16, colb], mrows_v[b, pl.ds(16, 16)])
            plsc.store_scatter(tbuf_v, [r32, colb], crows_v[b, pl.ds(0, 16)])
            plsc.store_scatter(tbuf_v, [r48, colb], crows_v[b, pl.ds(16, 16)])

        writes = []
        for db in range(DBLK):
            for bb in range(BBLK):
                w = pltpu.make_async_copy(
                    tbuf_v.at[pl.ds(db * 8, 8), pl.ds(bb * 128, 128)],
                    out_hbm.at[l, db, bb0 + bb, :, :], wsem)
                w.start()
                writes.append(w)
        for c in writes:
            c.wait()
        return ()

    lax.fori_loop(0, MAX_LEN, chunk, (), unroll=False)


@jax.jit
def _run(mid_idxT, cat_idxT, mid_table, cat_table):
    mesh = plsc.VectorSubcoreMesh(core_axis_name="c", subcore_axis_name="s")
    f = pl.kernel(
        _body,
        out_type=jax.ShapeDtypeStruct(
            (MAX_LEN, DBLK, BATCH // 128, 8, 128), jnp.float32),
        mesh=mesh,
        scratch_types=[
            pltpu.VMEM((BW,), jnp.int32),
            pltpu.VMEM((BW,), jnp.int32),
            pltpu.VMEM((BW, EMBED_DIM), jnp.float32),
            pltpu.VMEM((BW, EMBED_DIM), jnp.float32),
            pltpu.VMEM((D2, BW + 1), jnp.float32),
            pltpu.SemaphoreType.DMA,
            pltpu.SemaphoreType.DMA,
        ],
        compiler_params=pltpu.CompilerParams(use_tc_tiling_on_sc=False,
                                             needs_layout_passes=False,
                                             disable_bounds_checks=True),
    )
    return f(mid_idxT, cat_idxT, mid_table, cat_table)


def kernel(mid_his_input, cat_his_input, mid_table, cat_table):
    mid_idxT = mid_his_input.astype(jnp.int32).T
    cat_idxT = cat_his_input.astype(jnp.int32).T
    out5 = _run(mid_idxT, cat_idxT, mid_table, cat_table)
    t = jnp.transpose(out5, (2, 4, 0, 1, 3))
    return t.reshape(BATCH, MAX_LEN, D2)
